# baseline (speedup 1.0000x reference)
import functools
import math

import numpy as np
import jax
import jax.numpy as jnp
from jax import lax
from jax.experimental import pallas as pl
from jax.experimental.pallas import tpu as pltpu
from jax.experimental.pallas import tpu_sc as plsc

D_MODEL = 256
N_HEADS = 8
N_LEVELS = 4
N_POINTS = 4
SHAPES = ((64, 64), (32, 32), (16, 16), (8, 8))
LEN_IN = sum(h * w for h, w in SHAPES)
LVL_START = tuple(np.concatenate([[0], np.cumsum([h * w for h, w in SHAPES])[:-1]]).tolist())


def _matmul_kernel(x_ref, w_ref, b_ref, o_ref):
    acc = jnp.dot(x_ref[...], w_ref[...],
                  preferred_element_type=jnp.float32) + b_ref[...]
    o_ref[...] = acc.astype(o_ref.dtype)


def _pallas_matmul(x, w_t, b, out_dtype=jnp.float32):
    R, K = x.shape
    C = w_t.shape[1]
    BR = 1088
    return pl.pallas_call(
        _matmul_kernel,
        grid=(R // BR,),
        in_specs=[
            pl.BlockSpec((BR, K), lambda i: (i, 0)),
            pl.BlockSpec((K, C), lambda i: (0, 0)),
            pl.BlockSpec((1, C), lambda i: (0, 0)),
        ],
        out_specs=pl.BlockSpec((BR, C), lambda i: (i, 0)),
        out_shape=jax.ShapeDtypeStruct((R, C), out_dtype),
    )(x, w_t, b.reshape(1, C))


def _prep_kernel(q_ref, wx_ref, wy_ref, wa_ref, bx_ref, by_ref, ba_ref,
                 rp_ref, wl_ref, hl_ref, gmask_ref, basel_ref,
                 ml_ref, idx_ref, w_ref, *, blocks_per_n):
    n = pl.program_id(0) // blocks_per_n
    q = q_ref[...]
    offx = jnp.dot(q, wx_ref[...], preferred_element_type=jnp.float32) + bx_ref[...]
    offy = jnp.dot(q, wy_ref[...], preferred_element_type=jnp.float32) + by_ref[...]
    logit = jnp.dot(q, wa_ref[...], preferred_element_type=jnp.float32) + ba_ref[...]
    e = jnp.exp(logit - jnp.max(logit, axis=-1, keepdims=True))
    gs = jnp.dot(e, gmask_ref[...], preferred_element_type=jnp.float32)
    attnw = e / gs
    wl = wl_ref[...]
    hl = hl_ref[...]
    rp = rp_ref[...]
    bq = rp.shape[0]
    rx16 = jnp.concatenate(
        [jnp.broadcast_to(rp[:, 2 * l:2 * l + 1], (bq, 4)) for l in range(4)],
        axis=1)
    ry16 = jnp.concatenate(
        [jnp.broadcast_to(rp[:, 2 * l + 1:2 * l + 2], (bq, 4)) for l in range(4)],
        axis=1)
    rx = jnp.concatenate([rx16] * 8, axis=1)
    ry = jnp.concatenate([ry16] * 8, axis=1)
    gx = rx * wl + offx - 0.5
    gy = ry * hl + offy - 0.5
    x0 = jnp.floor(gx)
    y0 = jnp.floor(gy)
    fx = gx - x0
    fy = gy - y0
    x1 = x0 + 1.0
    y1 = y0 + 1.0
    vx0 = (x0 >= 0.0) & (x0 <= wl - 1.0)
    vx1 = (x1 >= 0.0) & (x1 <= wl - 1.0)
    vy0 = (y0 >= 0.0) & (y0 <= hl - 1.0)
    vy1 = (y1 >= 0.0) & (y1 <= hl - 1.0)
    xc0 = jnp.clip(x0, 0.0, wl - 1.0).astype(jnp.int32)
    xc1 = jnp.clip(x1, 0.0, wl - 1.0).astype(jnp.int32)
    yc0 = jnp.clip(y0, 0.0, hl - 1.0).astype(jnp.int32)
    yc1 = jnp.clip(y1, 0.0, hl - 1.0).astype(jnp.int32)
    wi = wl.astype(jnp.int32)
    base = basel_ref[...] + n * LEN_IN
    r0 = ((base + yc0 * wi) << 3) + ml_ref[...]
    r1 = ((base + yc1 * wi) << 3) + ml_ref[...]
    x0i = xc0 << 3
    x1i = xc1 << 3
    idx_ref[...] = jnp.concatenate(
        [r0 + x0i, r0 + x1i, r1 + x0i, r1 + x1i], axis=1)
    wx0 = 1.0 - fx
    wy0 = 1.0 - fy
    w_ref[...] = jnp.concatenate(
        [attnw * wx0 * wy0 * (vx0 & vy0).astype(jnp.float32),
         attnw * fx * wy0 * (vx1 & vy0).astype(jnp.float32),
         attnw * wx0 * fy * (vx0 & vy1).astype(jnp.float32),
         attnw * fx * fy * (vx1 & vy1).astype(jnp.float32)], axis=1)


def _sampling_prep(query, reference_points, W_off, b_off, W_attn, b_attn):
    N, Lq, C = query.shape
    M, L, P = N_HEADS, N_LEVELS, N_POINTS
    NQ = N * Lq
    BQ = 544
    blocks_per_n = Lq // BQ
    q2 = query.reshape(NQ, C)
    lanes = np.arange(M * L * P)
    l_of = (lanes // P) % L
    m_of = lanes // (L * P)
    w_np = np.array([s[1] for s in SHAPES], np.float32)[l_of]
    h_np = np.array([s[0] for s in SHAPES], np.float32)[l_of]
    base_np = np.array(LVL_START, np.int64)[l_of].astype(np.int32)
    ml_np = m_of.astype(np.int32)
    gmask = ((lanes[:, None] // (L * P)) == (lanes[None, :] // (L * P))).astype(np.float32)

    rp8 = reference_points.reshape(NQ, 8)

    out_shapes = [jax.ShapeDtypeStruct((NQ, 512), jnp.int32),
                  jax.ShapeDtypeStruct((NQ, 512), jnp.float32)]
    row_spec = pl.BlockSpec((BQ, 128), lambda i: (i, 0))
    out_spec = pl.BlockSpec((BQ, 512), lambda i: (i, 0))
    full_spec = lambda r: pl.BlockSpec((r, 128), lambda i: (0, 0))
    return pl.pallas_call(
        functools.partial(_prep_kernel, blocks_per_n=blocks_per_n),
        grid=(NQ // BQ,),
        in_specs=[
            pl.BlockSpec((BQ, 256), lambda i: (i, 0)),
            full_spec(256), full_spec(256), full_spec(256),
            full_spec(1), full_spec(1), full_spec(1),
            pl.BlockSpec((BQ, 8), lambda i: (i, 0)),
            full_spec(1), full_spec(1),
            full_spec(128),
            full_spec(1),
            full_spec(1),
        ],
        out_specs=[out_spec, out_spec],
        out_shape=out_shapes,
    )(q2,
      W_off.T[:, 0::2], W_off.T[:, 1::2], W_attn.T,
      b_off[0::2].reshape(1, 128), b_off[1::2].reshape(1, 128),
      b_attn.reshape(1, 128),
      rp8,
      jnp.asarray(w_np).reshape(1, 128), jnp.asarray(h_np).reshape(1, 128),
      jnp.asarray(gmask),
      jnp.asarray(base_np).reshape(1, 128),
      jnp.asarray(ml_np).reshape(1, 128))


def _sc_gather_blend(val_bf, iw2, QH, CQ):
    NC, NS = 2, 16
    NW = NC * NS
    chunks = QH // CQ
    cpw = chunks // NW
    G = CQ * 64

    mesh = plsc.VectorSubcoreMesh(core_axis_name="c", subcore_axis_name="s")

    @functools.partial(
        pl.kernel, mesh=mesh,
        out_type=jax.ShapeDtypeStruct((QH, 2, 16), jnp.float32),
        compiler_params=pltpu.CompilerParams(needs_layout_passes=False,
                                             use_tc_tiling_on_sc=False),
        scratch_types=[
            [pltpu.VMEM((2 * G,), jnp.int32)] * 2,
            [pltpu.VMEM((G, 32), jnp.bfloat16)] * 2,
            [pltpu.VMEM((CQ, 2, 16), jnp.float32)] * 2,
            [pltpu.SemaphoreType.DMA] * 2,
            [pltpu.SemaphoreType.DMA] * 2,
            [pltpu.SemaphoreType.DMA] * 2,
        ],
    )
    def sc_body(val_hbm, iw_hbm, out_hbm, iw_v, g_v, o_v,
                sem_i, sem_g, sem_o):
        wid = lax.axis_index("s") * NC + lax.axis_index("c")
        T = cpw

        def start_iw(t, b):
            pltpu.async_copy(iw_hbm.at[wid * T + t], iw_v[b], sem_i[b])

        def wait_iw(b):
            pltpu.make_async_copy(iw_hbm.at[0], iw_v[b], sem_i[b]).wait()

        def start_gather(b):
            pltpu.async_copy(val_hbm.at[iw_v[b].at[pl.ds(0, G)]], g_v[b],
                             sem_g[b])

        def wait_gather(b):
            pltpu.make_async_copy(val_hbm.at[iw_v[b].at[pl.ds(0, G)]],
                                  g_v[b], sem_g[b]).wait()

        def start_out(t, b):
            pltpu.async_copy(o_v[b],
                             out_hbm.at[pl.ds((wid * T + t) * CQ, CQ)],
                             sem_o[b])

        def wait_out(b):
            pltpu.make_async_copy(o_v[b], out_hbm.at[pl.ds(0, CQ)],
                                  sem_o[b]).wait()

        start_iw(0, 0)
        start_iw(1, 1)
        wait_iw(0)
        start_gather(0)

        def loop_body(t2, carry):
            for b in (0, 1):
                t = t2 * 2 + b

                @pl.when(t + 1 < T)
                def _():
                    wait_iw(1 - b)
                    start_gather(1 - b)

                wait_gather(b)

                @pl.when(t >= 2)
                def _():
                    wait_out(b)

                def qh_body(j, carry2):
                    acc0 = jnp.zeros((16,), jnp.float32)
                    acc1 = jnp.zeros((16,), jnp.float32)
                    base = (j // 8) * 512 + (j % 8) * 16
                    bf = jnp.full((16,), base, jnp.int32)
                    for c in range(4):
                        for si in range(16):
                            r = base + c * 128 + si
                            ws = plsc.bitcast(
                        plsc.load_gather(iw_v[b], [bf + (G + c * 128 + si)]),
                        jnp.float32)
                            lo, hi = plsc.unpack(
                                g_v[b][r], format=plsc.PackFormat.INTERLEAVED)
                            acc0 = acc0 + ws * lo
                            acc1 = acc1 + ws * hi
                    o_v[b][j, 0] = acc0
                    o_v[b][j, 1] = acc1
                    return carry2

                lax.fori_loop(0, CQ, qh_body, 0)

                @pl.when(t + 2 < T)
                def _():
                    start_iw(t + 2, b)

                start_out(t, b)
            return carry

        lax.fori_loop(0, T // 2, loop_body, 0)
        wait_out(0)
        wait_out(1)

    return sc_body(val_bf, iw2)


def kernel(query, reference_points, input_flatten, input_spatial_shapes,
           input_level_start_index, W_value, b_value, W_off, b_off,
           W_attn, b_attn, W_out, b_out):
    N, Lq, C = query.shape
    M, L, P = N_HEADS, N_LEVELS, N_POINTS
    D = C // M
    NQ = N * Lq
    QH = NQ * M

    perm = np.arange(256).reshape(N_HEADS, 2, 16).transpose(0, 2, 1).reshape(-1)
    val_bf = _pallas_matmul(input_flatten.reshape(N * LEN_IN, C),
                            W_value.T[:, perm], b_value[perm],
                            out_dtype=jnp.bfloat16).reshape(-1, 32)

    idx512, w512 = _sampling_prep(
        query, reference_points, W_off, b_off, W_attn, b_attn)

    CQ = 40
    chunks = QH // CQ
    G = CQ * 64
    iw2 = jnp.concatenate(
        [idx512.reshape(chunks, G),
         jax.lax.bitcast_convert_type(w512.reshape(chunks, G), jnp.int32)],
        axis=1)
    out_rows = _sc_gather_blend(val_bf, iw2, QH, CQ)

    out = out_rows.reshape(NQ, C)
    return _pallas_matmul(out, W_out.T, b_out).reshape(N, Lq, C)


# speedup vs baseline: 1.0539x; 1.0539x over previous
import functools
import math

import numpy as np
import jax
import jax.numpy as jnp
from jax import lax
from jax.experimental import pallas as pl
from jax.experimental.pallas import tpu as pltpu
from jax.experimental.pallas import tpu_sc as plsc

D_MODEL = 256
N_HEADS = 8
N_LEVELS = 4
N_POINTS = 4
SHAPES = ((64, 64), (32, 32), (16, 16), (8, 8))
LEN_IN = sum(h * w for h, w in SHAPES)
LVL_START = tuple(np.concatenate([[0], np.cumsum([h * w for h, w in SHAPES])[:-1]]).tolist())


def _matmul_kernel(x_ref, w_ref, b_ref, o_ref):
    acc = jnp.dot(x_ref[...], w_ref[...],
                  preferred_element_type=jnp.float32) + b_ref[...]
    o_ref[...] = acc.astype(o_ref.dtype)


def _pallas_matmul(x, w_t, b, out_dtype=jnp.float32):
    R, K = x.shape
    C = w_t.shape[1]
    BR = 1088
    return pl.pallas_call(
        _matmul_kernel,
        grid=(R // BR,),
        in_specs=[
            pl.BlockSpec((BR, K), lambda i: (i, 0)),
            pl.BlockSpec((K, C), lambda i: (0, 0)),
            pl.BlockSpec((1, C), lambda i: (0, 0)),
        ],
        out_specs=pl.BlockSpec((BR, C), lambda i: (i, 0)),
        out_shape=jax.ShapeDtypeStruct((R, C), out_dtype),
    )(x, w_t, b.reshape(1, C))


def _prep_kernel(q_ref, wx_ref, wy_ref, wa_ref, bx_ref, by_ref, ba_ref,
                 rx_ref, ry_ref, wl_ref, hl_ref, gmask_ref, basel_ref,
                 ml_ref, idx_ref, w_ref, *, blocks_per_n):
    n = pl.program_id(0) // blocks_per_n
    q = q_ref[...]
    offx = jnp.dot(q, wx_ref[...], preferred_element_type=jnp.float32) + bx_ref[...]
    offy = jnp.dot(q, wy_ref[...], preferred_element_type=jnp.float32) + by_ref[...]
    logit = jnp.dot(q, wa_ref[...], preferred_element_type=jnp.float32) + ba_ref[...]
    e = jnp.exp(logit - jnp.max(logit, axis=-1, keepdims=True))
    gs = jnp.dot(e, gmask_ref[...], preferred_element_type=jnp.float32)
    attnw = e / gs
    wl = wl_ref[...]
    hl = hl_ref[...]
    gx = rx_ref[...] * wl + offx - 0.5
    gy = ry_ref[...] * hl + offy - 0.5
    x0 = jnp.floor(gx)
    y0 = jnp.floor(gy)
    fx = gx - x0
    fy = gy - y0
    x1 = x0 + 1.0
    y1 = y0 + 1.0
    vx0 = (x0 >= 0.0) & (x0 <= wl - 1.0)
    vx1 = (x1 >= 0.0) & (x1 <= wl - 1.0)
    vy0 = (y0 >= 0.0) & (y0 <= hl - 1.0)
    vy1 = (y1 >= 0.0) & (y1 <= hl - 1.0)
    xc0 = jnp.clip(x0, 0.0, wl - 1.0).astype(jnp.int32)
    xc1 = jnp.clip(x1, 0.0, wl - 1.0).astype(jnp.int32)
    yc0 = jnp.clip(y0, 0.0, hl - 1.0).astype(jnp.int32)
    yc1 = jnp.clip(y1, 0.0, hl - 1.0).astype(jnp.int32)
    wi = wl.astype(jnp.int32)
    base = basel_ref[...] + n * LEN_IN
    r0 = ((base + yc0 * wi) << 3) + ml_ref[...]
    r1 = ((base + yc1 * wi) << 3) + ml_ref[...]
    x0i = xc0 << 3
    x1i = xc1 << 3
    idx_ref[...] = jnp.concatenate(
        [r0 + x0i, r0 + x1i, r1 + x0i, r1 + x1i], axis=1)
    wx0 = 1.0 - fx
    wy0 = 1.0 - fy
    w_ref[...] = jnp.concatenate(
        [attnw * wx0 * wy0 * (vx0 & vy0).astype(jnp.float32),
         attnw * fx * wy0 * (vx1 & vy0).astype(jnp.float32),
         attnw * wx0 * fy * (vx0 & vy1).astype(jnp.float32),
         attnw * fx * fy * (vx1 & vy1).astype(jnp.float32)], axis=1)


def _sampling_prep(query, reference_points, W_off, b_off, W_attn, b_attn):
    N, Lq, C = query.shape
    M, L, P = N_HEADS, N_LEVELS, N_POINTS
    NQ = N * Lq
    BQ = 544
    blocks_per_n = Lq // BQ
    q2 = query.reshape(NQ, C)
    lanes = np.arange(M * L * P)
    l_of = (lanes // P) % L
    m_of = lanes // (L * P)
    w_np = np.array([s[1] for s in SHAPES], np.float32)[l_of]
    h_np = np.array([s[0] for s in SHAPES], np.float32)[l_of]
    base_np = np.array(LVL_START, np.int64)[l_of].astype(np.int32)
    ml_np = m_of.astype(np.int32)
    gmask = ((lanes[:, None] // (L * P)) == (lanes[None, :] // (L * P))).astype(np.float32)

    rx = jnp.tile(jnp.repeat(reference_points[..., 0], P, axis=-1), (1, 1, M)).reshape(NQ, 128)
    ry = jnp.tile(jnp.repeat(reference_points[..., 1], P, axis=-1), (1, 1, M)).reshape(NQ, 128)

    out_shapes = [jax.ShapeDtypeStruct((NQ, 512), jnp.int32),
                  jax.ShapeDtypeStruct((NQ, 512), jnp.float32)]
    row_spec = pl.BlockSpec((BQ, 128), lambda i: (i, 0))
    out_spec = pl.BlockSpec((BQ, 512), lambda i: (i, 0))
    full_spec = lambda r: pl.BlockSpec((r, 128), lambda i: (0, 0))
    return pl.pallas_call(
        functools.partial(_prep_kernel, blocks_per_n=blocks_per_n),
        grid=(NQ // BQ,),
        in_specs=[
            pl.BlockSpec((BQ, 256), lambda i: (i, 0)),
            full_spec(256), full_spec(256), full_spec(256),
            full_spec(1), full_spec(1), full_spec(1),
            row_spec, row_spec,
            full_spec(1), full_spec(1),
            full_spec(128),
            full_spec(1),
            full_spec(1),
        ],
        out_specs=[out_spec, out_spec],
        out_shape=out_shapes,
    )(q2,
      W_off.T[:, 0::2], W_off.T[:, 1::2], W_attn.T,
      b_off[0::2].reshape(1, 128), b_off[1::2].reshape(1, 128),
      b_attn.reshape(1, 128),
      rx, ry,
      jnp.asarray(w_np).reshape(1, 128), jnp.asarray(h_np).reshape(1, 128),
      jnp.asarray(gmask),
      jnp.asarray(base_np).reshape(1, 128),
      jnp.asarray(ml_np).reshape(1, 128))


def _sc_gather_blend(val_bf, idx2, w2, QH, CQ):
    NC, NS = 2, 16
    NW = NC * NS
    chunks = QH // CQ
    cpw = chunks // NW
    G = CQ * 64

    mesh = plsc.VectorSubcoreMesh(core_axis_name="c", subcore_axis_name="s")

    @functools.partial(
        pl.kernel, mesh=mesh,
        out_type=jax.ShapeDtypeStruct((QH, 2, 16), jnp.float32),
        compiler_params=pltpu.CompilerParams(needs_layout_passes=False,
                                             use_tc_tiling_on_sc=False),
        scratch_types=[
            [pltpu.VMEM((G,), jnp.int32)] * 2,
            [pltpu.VMEM((G,), jnp.float32)] * 2,
            [pltpu.VMEM((G, 32), jnp.bfloat16)] * 2,
            [pltpu.VMEM((CQ, 2, 16), jnp.float32)] * 2,
            [pltpu.SemaphoreType.DMA] * 2,
            [pltpu.SemaphoreType.DMA] * 2,
            [pltpu.SemaphoreType.DMA] * 2,
            [pltpu.SemaphoreType.DMA] * 2,
        ],
    )
    def sc_body(val_hbm, idx_hbm, w_hbm, out_hbm, idx_v, w_v, g_v, o_v,
                sem_i, sem_w, sem_g, sem_o):
        wid = lax.axis_index("s") * NC + lax.axis_index("c")
        T = cpw

        def start_i(t, b):
            pltpu.async_copy(idx_hbm.at[wid * T + t], idx_v[b], sem_i[b])

        def start_w(t, b):
            pltpu.async_copy(w_hbm.at[wid * T + t], w_v[b], sem_w[b])

        def wait_iw(b):
            pltpu.make_async_copy(idx_hbm.at[0], idx_v[b], sem_i[b]).wait()
            pltpu.make_async_copy(w_hbm.at[0], w_v[b], sem_w[b]).wait()

        def start_gather(b):
            pltpu.async_copy(val_hbm.at[idx_v[b]], g_v[b], sem_g[b])

        def wait_gather(b):
            pltpu.make_async_copy(val_hbm.at[idx_v[b]], g_v[b],
                                  sem_g[b]).wait()

        def start_out(t, b):
            pltpu.async_copy(o_v[b],
                             out_hbm.at[pl.ds((wid * T + t) * CQ, CQ)],
                             sem_o[b])

        def wait_out(b):
            pltpu.make_async_copy(o_v[b], out_hbm.at[pl.ds(0, CQ)],
                                  sem_o[b]).wait()

        start_i(0, 0)
        start_w(0, 0)
        start_i(1, 1)
        start_w(1, 1)
        wait_iw(0)
        start_gather(0)

        def loop_body(t2, carry):
            for b in (0, 1):
                t = t2 * 2 + b

                @pl.when(t + 1 < T)
                def _():
                    wait_iw(1 - b)
                    start_gather(1 - b)

                wait_gather(b)

                @pl.when(t + 2 < T)
                def _():
                    start_i(t + 2, b)

                @pl.when(t >= 2)
                def _():
                    wait_out(b)

                def qh_body(j, carry2):
                    acc0 = jnp.zeros((16,), jnp.float32)
                    acc1 = jnp.zeros((16,), jnp.float32)
                    base = (j // 8) * 512 + (j % 8) * 16
                    bf = jnp.full((16,), base, jnp.int32)
                    for c in range(4):
                        for si in range(16):
                            r = base + c * 128 + si
                            ws = plsc.load_gather(w_v[b], [bf + (c * 128 + si)])
                            lo, hi = plsc.unpack(
                                g_v[b][r], format=plsc.PackFormat.INTERLEAVED)
                            acc0 = acc0 + ws * lo
                            acc1 = acc1 + ws * hi
                    o_v[b][j, 0] = acc0
                    o_v[b][j, 1] = acc1
                    return carry2

                lax.fori_loop(0, CQ, qh_body, 0)

                @pl.when(t + 2 < T)
                def _():
                    start_w(t + 2, b)

                start_out(t, b)
            return carry

        lax.fori_loop(0, T // 2, loop_body, 0)
        wait_out(0)
        wait_out(1)

    return sc_body(val_bf, idx2, w2)


def kernel(query, reference_points, input_flatten, input_spatial_shapes,
           input_level_start_index, W_value, b_value, W_off, b_off,
           W_attn, b_attn, W_out, b_out):
    N, Lq, C = query.shape
    M, L, P = N_HEADS, N_LEVELS, N_POINTS
    D = C // M
    NQ = N * Lq
    QH = NQ * M

    perm = np.arange(256).reshape(N_HEADS, 2, 16).transpose(0, 2, 1).reshape(-1)
    val_bf = _pallas_matmul(input_flatten.reshape(N * LEN_IN, C),
                            W_value.T[:, perm], b_value[perm],
                            out_dtype=jnp.bfloat16).reshape(-1, 32)

    idx512, w512 = _sampling_prep(
        query, reference_points, W_off, b_off, W_attn, b_attn)

    CQ = 40
    chunks = QH // CQ
    G = CQ * 64
    idx2 = idx512.reshape(chunks, G)
    w2 = w512.reshape(chunks, G)
    out_rows = _sc_gather_blend(val_bf, idx2, w2, QH, CQ)

    out = out_rows.reshape(NQ, C)
    return _pallas_matmul(out, W_out.T, b_out).reshape(N, Lq, C)
